# Initial kernel scaffold; baseline (speedup 1.0000x reference)
import jax
import jax.numpy as jnp
from jax.experimental import pallas as pl


def kernel(pred_large, pred_medium, pred_small, targets):
    raise NotImplementedError("write your pallas kernel here")


# speedup vs baseline: 1.4428x; 1.4428x over previous
import functools

import jax
import jax.numpy as jnp
import numpy as np
from jax import lax
from jax.experimental import pallas as pl
from jax.experimental.pallas import tpu as pltpu
from jax.experimental.pallas import tpu_sc as plsc

_ANCHORS = np.array([[[116.0, 90.0], [156.0, 198.0], [373.0, 326.0]],
                     [[30.0, 61.0], [62.0, 45.0], [59.0, 119.0]],
                     [[10.0, 13.0], [16.0, 30.0], [33.0, 23.0]]], dtype=np.float32)
_NUM_CLASSES = 25
_GRIDS = (13, 26, 52)
_NCH = 30
_NT = 512
_NC, _NS = 2, 16
_NW = _NC * _NS
_BPW = _NT // _NW

_SCALED = [(_ANCHORS[i] / np.float32(g)).astype(np.float32)
           for i, g in enumerate(_GRIDS)]


def _softplus(x):
    return jnp.log1p(jnp.exp(-jnp.abs(x))) + jnp.maximum(x, 0.0)


def _bce(logits, tgt):
    return tgt * _softplus(-logits) + (1.0 - tgt) * _softplus(logits)


_DENSE_ROWS = 507


def _dense_body(x_ref, o_ref):
    pid = pl.program_id(0)
    r = lax.broadcasted_iota(jnp.int32, (1, _DENSE_ROWS, 128), 1)
    c = lax.broadcasted_iota(jnp.int32, (1, _DENSE_ROWS, 128), 2)
    pos = (pid * _DENSE_ROWS + r) * 128 + c
    m = (pos % _NCH) == 4
    x = x_ref[...]
    part = jnp.sum(jnp.where(m, _softplus(x), 0.0))

    @pl.when(pid == 0)
    def _():
        o_ref[0, 0] = 0.0

    o_ref[0, 0] += 0.5 * part


def _dense_sum(pred):
    flat = pred.reshape(-1, _DENSE_ROWS, 128)
    g = flat.shape[0]
    return pl.pallas_call(
        _dense_body,
        grid=(g,),
        in_specs=[pl.BlockSpec((1, _DENSE_ROWS, 128), lambda i: (i, 0, 0))],
        out_specs=pl.BlockSpec(memory_space=pltpu.SMEM),
        out_shape=jax.ShapeDtypeStruct((1, 1), jnp.float32),
    )(flat)


def _sc_cells(ts_v, grid, scaled):
    gf = jnp.float32(grid)
    bx = ts_v[0, :].astype(jnp.int32)
    txf = ts_v[2, :] * gf
    tyf = ts_v[3, :] * gf
    twf = ts_v[4, :] * gf
    thf = ts_v[5, :] * gf
    gx = jnp.clip(txf.astype(jnp.int32), 0, grid - 1)
    gy = jnp.clip(tyf.astype(jnp.int32), 0, grid - 1)
    ious = []
    for k in range(3):
        aw = jnp.float32(scaled[k, 0])
        ah = jnp.float32(scaled[k, 1])
        inter = jnp.minimum(twf, aw) * jnp.minimum(thf, ah)
        union = twf * thf + aw * ah - inter + 1e-9
        ious.append(inter / union)
    best = jnp.where(ious[1] > ious[0], 1, 0)
    best = jnp.where(ious[2] > jnp.maximum(ious[0], ious[1]), 2, best)
    return ((bx * 3 + best) * grid + gy) * grid + gx


def _sc_gather(targets_t, p_l, p_m, p_s):
    mesh = plsc.VectorSubcoreMesh(core_axis_name="c", subcore_axis_name="s")
    out_type = [jax.ShapeDtypeStruct((_NW, 2 * _BPW, 128), jnp.float32)
                for _ in range(3)]

    @functools.partial(
        pl.kernel,
        out_type=out_type,
        mesh=mesh,
        scratch_types=[
            pltpu.VMEM((6, _BPW), jnp.float32),
            pltpu.VMEM((2 * _BPW,), jnp.int32),
            pltpu.VMEM((2 * _BPW, 128), jnp.float32),
            pltpu.SemaphoreType.DMA,
        ],
    )
    def k(tt, t_l, t_m, t_s, o_l, o_m, o_s, ts_v, ridx_v, rows_v, sem):
        wid = lax.axis_index("s") * _NC + lax.axis_index("c")
        base = wid * _BPW
        for j in range(6):
            pltpu.sync_copy(tt.at[j, pl.ds(base, _BPW)], ts_v.at[j])
        for grid, scaled, table, out in (
                (_GRIDS[0], _SCALED[0], t_l, o_l),
                (_GRIDS[1], _SCALED[1], t_m, o_m),
                (_GRIDS[2], _SCALED[2], t_s, o_s)):
            elem0 = _sc_cells(ts_v, grid, scaled) * _NCH
            r0 = lax.shift_right_logical(elem0, 7)
            r1 = lax.shift_right_logical(elem0 + (_NCH - 1), 7)
            ridx_v[pl.ds(0, _BPW)] = r0
            ridx_v[pl.ds(_BPW, _BPW)] = r1
            pltpu.async_copy(table.at[ridx_v], rows_v, sem).wait()
            pltpu.sync_copy(rows_v, out.at[wid])

    return k(targets_t, p_l, p_m, p_s)


def _scale_corr(tt, rows, grid, scaled):
    gf = jnp.float32(grid)
    bx = tt[0, :].astype(jnp.int32)
    cls = tt[1, :].astype(jnp.int32)
    txf = tt[2, :] * gf
    tyf = tt[3, :] * gf
    twf = tt[4, :] * gf
    thf = tt[5, :] * gf
    gx = jnp.clip(txf.astype(jnp.int32), 0, grid - 1)
    gy = jnp.clip(tyf.astype(jnp.int32), 0, grid - 1)
    ious = []
    for k in range(3):
        aw = jnp.float32(scaled[k, 0])
        ah = jnp.float32(scaled[k, 1])
        inter = jnp.minimum(twf, aw) * jnp.minimum(thf, ah)
        union = twf * thf + aw * ah - inter + 1e-9
        ious.append(inter / union)
    best = jnp.where(ious[1] > ious[0], 1, 0)
    best = jnp.where(ious[2] > jnp.maximum(ious[0], ious[1]), 2, best)
    aw = jnp.where(best == 0, scaled[0, 0],
                   jnp.where(best == 1, scaled[1, 0], scaled[2, 0]))
    ah = jnp.where(best == 0, scaled[0, 1],
                   jnp.where(best == 1, scaled[1, 1], scaled[2, 1]))
    cell = ((bx * 3 + best) * grid + gy) * grid + gx

    ii = lax.broadcasted_iota(jnp.int32, (_NT, _NT), 0)
    jj = lax.broadcasted_iota(jnp.int32, (_NT, _NT), 1)
    later = jj > ii
    winner = ~jnp.any((cell[:, None] == cell[None, :]) & later, axis=1)
    key2 = cell * 32 + cls
    clsrep = ~jnp.any((key2[:, None] == key2[None, :]) & later, axis=1)

    tx = txf - gx.astype(jnp.float32)
    ty = tyf - gy.astype(jnp.float32)
    tw = jnp.log(twf / aw + 1e-16)
    th = jnp.log(thf / ah + 1e-16)

    off = lax.bitwise_and(cell * _NCH, 127)
    lane = lax.broadcasted_iota(jnp.int32, (_NT, 2 * 128), 1)
    chan = [jnp.sum(jnp.where(lane == (off + ch)[:, None], rows, 0.0), axis=1)
            for ch in range(_NCH)]
    px, py, pw, ph, po = chan[:5]
    cls_dense = jnp.zeros((_NT,), jnp.float32)
    pc = jnp.zeros((_NT,), jnp.float32)
    for c in range(_NUM_CLASSES):
        v = chan[5 + c]
        cls_dense = cls_dense + _softplus(v)
        pc = pc + jnp.where(cls == c, v, 0.0)

    wterm = (5.0 * (_bce(px, tx) + _bce(py, ty) + (pw - tw) ** 2 + (ph - th) ** 2)
             + _softplus(-po) - 0.5 * _softplus(po)
             + cls_dense)
    flip = _softplus(-pc) - _softplus(pc)
    return (jnp.sum(jnp.where(winner, wterm, 0.0))
            + jnp.sum(jnp.where(clsrep, flip, 0.0)))


def _corr_body(tt_ref, gl_ref, gm_ref, gs_ref, o_ref):
    tt = tt_ref[...]
    total = (_scale_corr(tt, gl_ref[...], _GRIDS[0], _SCALED[0])
             + _scale_corr(tt, gm_ref[...], _GRIDS[1], _SCALED[1])
             + _scale_corr(tt, gs_ref[...], _GRIDS[2], _SCALED[2]))
    o_ref[0, 0] = total


def _corr_sum(targets_t, gl_t, gm_t, gs_t):
    return pl.pallas_call(
        _corr_body,
        out_specs=pl.BlockSpec(memory_space=pltpu.SMEM),
        out_shape=jax.ShapeDtypeStruct((1, 1), jnp.float32),
    )(targets_t, gl_t, gm_t, gs_t)


def kernel(pred_large, pred_medium, pred_small, targets):
    p_l = pred_large.reshape(-1, 128)
    p_m = pred_medium.reshape(-1, 128)
    p_s = pred_small.reshape(-1, 128)
    targets_t = targets.T

    d_l = _dense_sum(pred_large)
    d_m = _dense_sum(pred_medium)
    d_s = _dense_sum(pred_small)

    g_l, g_m, g_s = _sc_gather(targets_t, p_l, p_m, p_s)
    rows = [jnp.concatenate([g[:, :_BPW, :], g[:, _BPW:, :]], axis=2)
            .reshape(_NT, 256) for g in (g_l, g_m, g_s)]

    corr = _corr_sum(targets_t, *rows)
    return (d_l + d_m + d_s + corr)[0, 0]


# speedup vs baseline: 1.8202x; 1.2616x over previous
import functools

import jax
import jax.numpy as jnp
import numpy as np
from jax import lax
from jax.experimental import pallas as pl
from jax.experimental.pallas import tpu as pltpu
from jax.experimental.pallas import tpu_sc as plsc

_ANCHORS = np.array([[[116.0, 90.0], [156.0, 198.0], [373.0, 326.0]],
                     [[30.0, 61.0], [62.0, 45.0], [59.0, 119.0]],
                     [[10.0, 13.0], [16.0, 30.0], [33.0, 23.0]]], dtype=np.float32)
_NUM_CLASSES = 25
_GRIDS = (13, 26, 52)
_NCH = 30
_NT = 512
_NC, _NS = 2, 16
_NW = _NC * _NS
_BPW = _NT // _NW

_SCALED = [(_ANCHORS[i] / np.float32(g)).astype(np.float32)
           for i, g in enumerate(_GRIDS)]


def _softplus(x):
    return jnp.log1p(jnp.exp(-jnp.abs(x))) + jnp.maximum(x, 0.0)


def _bce(logits, tgt):
    return tgt * _softplus(-logits) + (1.0 - tgt) * _softplus(logits)


_DENSE_ROWS = 585

_DENSE_MASK = np.zeros((_DENSE_ROWS, 128), np.float32)
_DENSE_MASK.reshape(-1)[4::_NCH] = 0.5


def _dense_body(x_ref, m_ref, o_ref):
    pid = pl.program_id(0)
    part = jnp.sum(_softplus(x_ref[...]) * m_ref[...])

    @pl.when(pid == 0)
    def _():
        o_ref[0, 0] = 0.0

    o_ref[0, 0] += part


def _dense_sum(pred, mask):
    flat = pred.reshape(-1, _DENSE_ROWS, 128)
    g = flat.shape[0]
    return pl.pallas_call(
        _dense_body,
        grid=(g,),
        in_specs=[pl.BlockSpec((1, _DENSE_ROWS, 128), lambda i: (i, 0, 0)),
                  pl.BlockSpec((_DENSE_ROWS, 128), lambda i: (0, 0))],
        out_specs=pl.BlockSpec(memory_space=pltpu.SMEM),
        out_shape=jax.ShapeDtypeStruct((1, 1), jnp.float32),
    )(flat, mask)


def _sc_cells(ts_v, grid, scaled):
    gf = jnp.float32(grid)
    bx = ts_v[0, :].astype(jnp.int32)
    txf = ts_v[2, :] * gf
    tyf = ts_v[3, :] * gf
    twf = ts_v[4, :] * gf
    thf = ts_v[5, :] * gf
    gx = jnp.clip(txf.astype(jnp.int32), 0, grid - 1)
    gy = jnp.clip(tyf.astype(jnp.int32), 0, grid - 1)
    ious = []
    for k in range(3):
        aw = jnp.float32(scaled[k, 0])
        ah = jnp.float32(scaled[k, 1])
        inter = jnp.minimum(twf, aw) * jnp.minimum(thf, ah)
        union = twf * thf + aw * ah - inter + 1e-9
        ious.append(inter / union)
    best = jnp.where(ious[1] > ious[0], 1, 0)
    best = jnp.where(ious[2] > jnp.maximum(ious[0], ious[1]), 2, best)
    return ((bx * 3 + best) * grid + gy) * grid + gx


def _sc_gather(targets_t, p_l, p_m, p_s):
    mesh = plsc.VectorSubcoreMesh(core_axis_name="c", subcore_axis_name="s")
    out_type = [jax.ShapeDtypeStruct((_NW, 2 * _BPW, 128), jnp.float32)
                for _ in range(3)]

    @functools.partial(
        pl.kernel,
        out_type=out_type,
        mesh=mesh,
        scratch_types=[
            pltpu.VMEM((6, _BPW), jnp.float32),
            pltpu.VMEM((2 * _BPW,), jnp.int32),
            pltpu.VMEM((2 * _BPW, 128), jnp.float32),
            pltpu.SemaphoreType.DMA,
        ],
    )
    def k(tt, t_l, t_m, t_s, o_l, o_m, o_s, ts_v, ridx_v, rows_v, sem):
        wid = lax.axis_index("s") * _NC + lax.axis_index("c")
        base = wid * _BPW
        for j in range(6):
            pltpu.sync_copy(tt.at[j, pl.ds(base, _BPW)], ts_v.at[j])
        for grid, scaled, table, out in (
                (_GRIDS[0], _SCALED[0], t_l, o_l),
                (_GRIDS[1], _SCALED[1], t_m, o_m),
                (_GRIDS[2], _SCALED[2], t_s, o_s)):
            elem0 = _sc_cells(ts_v, grid, scaled) * _NCH
            r0 = lax.shift_right_logical(elem0, 7)
            r1 = lax.shift_right_logical(elem0 + (_NCH - 1), 7)
            ridx_v[pl.ds(0, _BPW)] = r0
            ridx_v[pl.ds(_BPW, _BPW)] = r1
            pltpu.async_copy(table.at[ridx_v], rows_v, sem).wait()
            pltpu.sync_copy(rows_v, out.at[wid])

    return k(targets_t, p_l, p_m, p_s)


def _scale_corr(tt, rows, grid, scaled):
    gf = jnp.float32(grid)
    bx = tt[0, :].astype(jnp.int32)
    cls = tt[1, :].astype(jnp.int32)
    txf = tt[2, :] * gf
    tyf = tt[3, :] * gf
    twf = tt[4, :] * gf
    thf = tt[5, :] * gf
    gx = jnp.clip(txf.astype(jnp.int32), 0, grid - 1)
    gy = jnp.clip(tyf.astype(jnp.int32), 0, grid - 1)
    ious = []
    for k in range(3):
        aw = jnp.float32(scaled[k, 0])
        ah = jnp.float32(scaled[k, 1])
        inter = jnp.minimum(twf, aw) * jnp.minimum(thf, ah)
        union = twf * thf + aw * ah - inter + 1e-9
        ious.append(inter / union)
    best = jnp.where(ious[1] > ious[0], 1, 0)
    best = jnp.where(ious[2] > jnp.maximum(ious[0], ious[1]), 2, best)
    aw = jnp.where(best == 0, scaled[0, 0],
                   jnp.where(best == 1, scaled[1, 0], scaled[2, 0]))
    ah = jnp.where(best == 0, scaled[0, 1],
                   jnp.where(best == 1, scaled[1, 1], scaled[2, 1]))
    cell = ((bx * 3 + best) * grid + gy) * grid + gx

    ii = lax.broadcasted_iota(jnp.int32, (_NT, _NT), 0)
    jj = lax.broadcasted_iota(jnp.int32, (_NT, _NT), 1)
    later = jj > ii
    winner = ~jnp.any((cell[:, None] == cell[None, :]) & later, axis=1)
    key2 = cell * 32 + cls
    clsrep = ~jnp.any((key2[:, None] == key2[None, :]) & later, axis=1)

    tx = txf - gx.astype(jnp.float32)
    ty = tyf - gy.astype(jnp.float32)
    tw = jnp.log(twf / aw + 1e-16)
    th = jnp.log(thf / ah + 1e-16)

    off = lax.bitwise_and(cell * _NCH, 127)
    lane = lax.broadcasted_iota(jnp.int32, (_NT, 2 * 128), 1)
    chan = [jnp.sum(jnp.where(lane == (off + ch)[:, None], rows, 0.0), axis=1)
            for ch in range(_NCH)]
    px, py, pw, ph, po = chan[:5]
    cls_dense = jnp.zeros((_NT,), jnp.float32)
    pc = jnp.zeros((_NT,), jnp.float32)
    for c in range(_NUM_CLASSES):
        v = chan[5 + c]
        cls_dense = cls_dense + _softplus(v)
        pc = pc + jnp.where(cls == c, v, 0.0)

    wterm = (5.0 * (_bce(px, tx) + _bce(py, ty) + (pw - tw) ** 2 + (ph - th) ** 2)
             + _softplus(-po) - 0.5 * _softplus(po)
             + cls_dense)
    flip = _softplus(-pc) - _softplus(pc)
    return (jnp.sum(jnp.where(winner, wterm, 0.0))
            + jnp.sum(jnp.where(clsrep, flip, 0.0)))


def _corr_body(tt_ref, gl_ref, gm_ref, gs_ref, o_ref):
    tt = tt_ref[...]
    total = (_scale_corr(tt, gl_ref[...], _GRIDS[0], _SCALED[0])
             + _scale_corr(tt, gm_ref[...], _GRIDS[1], _SCALED[1])
             + _scale_corr(tt, gs_ref[...], _GRIDS[2], _SCALED[2]))
    o_ref[0, 0] = total


def _corr_sum(targets_t, gl_t, gm_t, gs_t):
    return pl.pallas_call(
        _corr_body,
        out_specs=pl.BlockSpec(memory_space=pltpu.SMEM),
        out_shape=jax.ShapeDtypeStruct((1, 1), jnp.float32),
    )(targets_t, gl_t, gm_t, gs_t)


def kernel(pred_large, pred_medium, pred_small, targets):
    p_l = pred_large.reshape(-1, 128)
    p_m = pred_medium.reshape(-1, 128)
    p_s = pred_small.reshape(-1, 128)
    targets_t = targets.T

    mask = jnp.asarray(_DENSE_MASK)
    d_l = _dense_sum(pred_large, mask)
    d_m = _dense_sum(pred_medium, mask)
    d_s = _dense_sum(pred_small, mask)

    g_l, g_m, g_s = _sc_gather(targets_t, p_l, p_m, p_s)
    rows = [jnp.concatenate([g[:, :_BPW, :], g[:, _BPW:, :]], axis=2)
            .reshape(_NT, 256) for g in (g_l, g_m, g_s)]

    corr = _corr_sum(targets_t, *rows)
    return (d_l + d_m + d_s + corr)[0, 0]
